# Initial kernel scaffold; baseline (speedup 1.0000x reference)
import jax
import jax.numpy as jnp
from jax.experimental import pallas as pl


def kernel(ts, rho, grid0, grid1, grid2, grid3):
    raise NotImplementedError("write your pallas kernel here")


# speedup vs baseline: 4.7303x; 4.7303x over previous
import functools

import jax
import jax.numpy as jnp
from jax import lax
from jax.experimental import pallas as pl
from jax.experimental.pallas import tpu as pltpu
from jax.experimental.pallas import tpu_sc as plsc

DIM = 32
LEVELS = 4
H0, W0 = 64, 256
NC, NS, L = 2, 16, 16
NW = NC * NS
CHUNK = 128
NTAP = 4 * LEVELS
NDMA = NTAP * CHUNK // 128


def _sc_sample(tsf, rhof, table):
    P = tsf.shape[0]
    ppw = P // NW
    nchunks = ppw // CHUNK
    mesh = plsc.VectorSubcoreMesh(core_axis_name="c", subcore_axis_name="s")

    @functools.partial(
        pl.kernel,
        out_type=jax.ShapeDtypeStruct((P, LEVELS * DIM), jnp.float32),
        mesh=mesh,
        scratch_types=[
            pltpu.VMEM((CHUNK,), jnp.float32),
            pltpu.VMEM((CHUNK,), jnp.float32),
            pltpu.VMEM((NTAP * CHUNK,), jnp.int32),
            pltpu.VMEM((NTAP * CHUNK,), jnp.float32),
            pltpu.VMEM((NTAP * CHUNK, DIM), jnp.float32),
            pltpu.VMEM((CHUNK, LEVELS * DIM), jnp.float32),
            pltpu.SemaphoreType.DMA,
        ],
        compiler_params=pltpu.CompilerParams(use_tc_tiling_on_sc=False),
    )
    def k(ts_hbm, rho_hbm, tab_hbm, out_hbm,
          ts_v, rho_v, idx_v, w_v, rows_v, out_v, sem_g):
        wid = lax.axis_index("s") * NC + lax.axis_index("c")

        lane = lax.iota(jnp.int32, L)
        tvec = lane & 3
        lvec = lane >> 2
        wl_i = W0 << lvec
        hl_i = H0 << lvec
        wm1_f = (wl_i - 1).astype(jnp.float32)
        hm1_f = (hl_i - 1).astype(jnp.float32)
        wm2_i = wl_i - 2
        hm2_i = hl_i - 2
        tap_dx = tvec & 1
        tap_dy = tvec >> 1
        mask_x1 = tap_dx == 1
        mask_y1 = tap_dy == 1
        hw = H0 * W0
        addc = tap_dx + jnp.where(
            lvec == 0, 0, jnp.where(lvec == 1, hw,
                                    jnp.where(lvec == 2, 5 * hw, 21 * hw)))

        def chunk_body(ci, carry):
            base = wid * ppw + ci * CHUNK
            pltpu.sync_copy(ts_hbm.at[pl.ds(base, CHUNK)], ts_v)
            pltpu.sync_copy(rho_hbm.at[pl.ds(base, CHUNK)], rho_v)

            def grp_body(g, carry2):
                off = g * L
                tsv = ts_v[pl.ds(off, L)]
                rhv = rho_v[pl.ds(off, L)]
                gx = 2.0 * jnp.minimum(jnp.maximum(rhv, 0.0), 1.0) - 1.0
                gy = 2.0 * jnp.minimum(jnp.maximum(tsv, 0.0), 1.0) - 1.0
                xsv = (gx + 1.0) * 0.5
                ysv = (gy + 1.0) * 0.5
                for kk in range(L):
                    x = jnp.broadcast_to(xsv[kk], (L,)) * wm1_f
                    y = jnp.broadcast_to(ysv[kk], (L,)) * hm1_f
                    x0 = jnp.minimum(x.astype(jnp.int32), wm2_i)
                    y0 = jnp.minimum(y.astype(jnp.int32), hm2_i)
                    wx = x - x0.astype(jnp.float32)
                    wy = y - y0.astype(jnp.float32)
                    idx = (y0 + tap_dy) * wl_i + x0 + addc
                    wxx = jnp.where(mask_x1, wx, 1.0 - wx)
                    wyy = jnp.where(mask_y1, wy, 1.0 - wy)
                    poff = (off + kk) * NTAP
                    idx_v[pl.ds(poff, NTAP)] = idx
                    w_v[pl.ds(poff, NTAP)] = wxx * wyy
                return carry2

            lax.fori_loop(0, CHUNK // L, grp_body, 0)

            cps = []
            for j in range(NDMA):
                cps.append(pltpu.async_copy(
                    tab_hbm.at[idx_v.at[pl.ds(j * 128, 128)]],
                    rows_v.at[pl.ds(j * 128, 128)], sem_g))
            for cp in cps:
                cp.wait()

            def pt_body(p, carry2):
                wvec = w_v[pl.ds(p * NTAP, NTAP)]
                for l in range(LEVELS):
                    for half in range(2):
                        hoff = half * L
                        acc = None
                        for t in range(4):
                            j = 4 * l + t
                            v = rows_v[p * NTAP + j, pl.ds(hoff, L)]
                            term = v * jnp.broadcast_to(wvec[j], (L,))
                            acc = term if acc is None else acc + term
                        out_v[p, pl.ds(l * DIM + hoff, L)] = acc
                return carry2

            lax.fori_loop(0, CHUNK, pt_body, 0)

            pltpu.sync_copy(out_v, out_hbm.at[pl.ds(base, CHUNK)])
            return carry

        lax.fori_loop(0, nchunks, chunk_body, 0)

    return k(tsf, rhof, table)


def kernel(ts, rho, grid0, grid1, grid2, grid3):
    B, N = ts.shape
    P = B * N
    tables = []
    for g in (grid0, grid1, grid2, grid3):
        c, h, w = g.shape[1], g.shape[2], g.shape[3]
        tables.append(g[0].reshape(c, h * w).T)
    table = jnp.concatenate(tables, axis=0)
    out = _sc_sample(ts.reshape(P), rho.reshape(P), table)
    return out.reshape(B, N, LEVELS * DIM)
